# baseline (speedup 1.0000x reference)
import functools
import math

import jax
import jax.numpy as jnp
from jax import lax
from jax.experimental import pallas as pl
from jax.experimental.pallas import tpu as pltpu
from jax.experimental.pallas import tpu_sc as plsc

D_MODEL = 64
SCALE = math.sqrt(D_MODEL)

_info = plsc.get_sparse_core_info()
NC = _info.num_cores
NS = _info.num_subcores
L = _info.num_lanes
NW = NC * NS

B_TOTAL = 4096 * 200
BPW = B_TOTAL // NW
IDX_W = 128
IDX_ROWS = BPW // IDX_W
GPB = 4
BLK = GPB * IDX_W
NBLK = BPW // BLK
VPR = D_MODEL // L


def _make_kernel():
  mesh = plsc.VectorSubcoreMesh(core_axis_name="c", subcore_axis_name="s")

  @functools.partial(
      pl.kernel,
      mesh=mesh,
      out_type=jax.ShapeDtypeStruct((NW, BPW, D_MODEL), jnp.float32),
      scratch_types=[
          pltpu.VMEM((IDX_ROWS, IDX_W), jnp.int32),
          pltpu.VMEM((BLK, D_MODEL), jnp.float32),
          pltpu.SemaphoreType.DMA,
      ],
      compiler_params=pltpu.CompilerParams(use_tc_tiling_on_sc=False),
  )
  def emb_kernel(x_hbm, table_hbm, out_hbm, idx_v, buf_v, sem):
    wid = lax.axis_index("s") * NC + lax.axis_index("c")
    pltpu.sync_copy(x_hbm.at[wid], idx_v)

    def blk_body(blk, carry):
      copies = []
      for j in range(GPB):
        c = pltpu.async_copy(
            table_hbm.at[idx_v.at[blk * GPB + j]],
            buf_v.at[pl.ds(j * IDX_W, IDX_W)],
            sem,
        )
        copies.append(c)
      for c in copies:
        c.wait()

      def row_body(r, c2):
        for cc in range(VPR):
          sl = pl.ds(cc * L, L)
          buf_v[r, sl] = buf_v[r, sl] * SCALE
        return c2

      lax.fori_loop(0, BLK, row_body, 0)

      pltpu.sync_copy(buf_v, out_hbm.at[wid, pl.ds(blk * BLK, BLK)])
      return carry

    lax.fori_loop(0, NBLK, blk_body, 0)

  return emb_kernel


_emb_kernel = _make_kernel()


@jax.jit
def kernel(x, table):
  x32 = x.reshape(NW, IDX_ROWS, IDX_W).astype(jnp.int32)
  out = _emb_kernel(x32, table)
  return out.reshape(4096, 200, D_MODEL)


# speedup vs baseline: 1.1168x; 1.1168x over previous
import functools
import math

import jax
import jax.numpy as jnp
from jax import lax
from jax.experimental import pallas as pl
from jax.experimental.pallas import tpu as pltpu
from jax.experimental.pallas import tpu_sc as plsc

D_MODEL = 64
SCALE = math.sqrt(D_MODEL)

_info = plsc.get_sparse_core_info()
NC = _info.num_cores
NS = _info.num_subcores
L = _info.num_lanes
NW = NC * NS

B_TOTAL = 4096 * 200
BPW = B_TOTAL // NW
IDX_W = 128
IDX_ROWS = BPW // IDX_W
GPB = 4
BLK = GPB * IDX_W
NBLK = BPW // BLK
VPR = D_MODEL // L


def _make_kernel():
  mesh = plsc.VectorSubcoreMesh(core_axis_name="c", subcore_axis_name="s")

  @functools.partial(
      pl.kernel,
      mesh=mesh,
      out_type=jax.ShapeDtypeStruct((NW, BPW, D_MODEL), jnp.float32),
      scratch_types=[
          pltpu.VMEM((IDX_ROWS, IDX_W), jnp.int32),
          pltpu.VMEM((BLK, D_MODEL), jnp.float32),
          pltpu.VMEM((BLK, D_MODEL), jnp.float32),
          pltpu.SemaphoreType.DMA,
          pltpu.SemaphoreType.DMA,
          pltpu.SemaphoreType.DMA,
          pltpu.SemaphoreType.DMA,
      ],
      compiler_params=pltpu.CompilerParams(use_tc_tiling_on_sc=False),
  )
  def emb_kernel(x_hbm, table_hbm, out_hbm, idx_v, buf0, buf1,
                 gsem0, gsem1, osem0, osem1):
    wid = lax.axis_index("s") * NC + lax.axis_index("c")
    pltpu.sync_copy(x_hbm.at[wid], idx_v)

    def fire_gathers(blk, buf, sem):
      return [
          pltpu.async_copy(
              table_hbm.at[idx_v.at[blk * GPB + j]],
              buf.at[pl.ds(j * IDX_W, IDX_W)],
              sem,
          )
          for j in range(GPB)
      ]

    def drain_gathers(buf, sem):
      for j in range(GPB):
        pltpu.make_async_copy(
            table_hbm.at[idx_v.at[0]],
            buf.at[pl.ds(j * IDX_W, IDX_W)],
            sem,
        ).wait()

    def scale(buf):
      @plsc.parallel_loop(0, BLK, step=1, unroll=8)
      def _(r):
        for cc in range(VPR):
          sl = pl.ds(cc * L, L)
          buf[r, sl] = buf[r, sl] * SCALE

    def fire_out(blk, buf, sem):
      return pltpu.async_copy(buf, out_hbm.at[wid, pl.ds(blk * BLK, BLK)],
                              sem)

    def drain_out(buf, sem):
      pltpu.make_async_copy(buf, out_hbm.at[wid, pl.ds(0, BLK)], sem).wait()

    fire_gathers(0, buf0, gsem0)

    def body(i, carry):
      b0 = 2 * i
      @pl.when(i > 0)
      def _():
        drain_out(buf1, osem1)
      fire_gathers(b0 + 1, buf1, gsem1)
      drain_gathers(buf0, gsem0)
      scale(buf0)
      fire_out(b0, buf0, osem0)
      drain_out(buf0, osem0)
      @pl.when(i < NBLK // 2 - 1)
      def _():
        fire_gathers(b0 + 2, buf0, gsem0)
      drain_gathers(buf1, gsem1)
      scale(buf1)
      fire_out(b0 + 1, buf1, osem1)
      return carry

    lax.fori_loop(0, NBLK // 2, body, 0)
    drain_out(buf1, osem1)

  return emb_kernel


_emb_kernel = _make_kernel()


@jax.jit
def kernel(x, table):
  x32 = x.reshape(NW, IDX_ROWS, IDX_W).astype(jnp.int32)
  out = _emb_kernel(x32, table)
  return out.reshape(4096, 200, D_MODEL)


# speedup vs baseline: 1.1901x; 1.0657x over previous
import functools
import math

import jax
import jax.numpy as jnp
from jax import lax
from jax.experimental import pallas as pl
from jax.experimental.pallas import tpu as pltpu
from jax.experimental.pallas import tpu_sc as plsc

D_MODEL = 64
SCALE = math.sqrt(D_MODEL)

_info = plsc.get_sparse_core_info()
NC = _info.num_cores
NS = _info.num_subcores
L = _info.num_lanes
NW = NC * NS

B = 4096
S = 200
IDX_W = 128
TI = B // IDX_W
N_UNITS = S * TI
UPW = N_UNITS // NW
VPR = D_MODEL // L
NBUF = 4
LOOK = 2


def _make_kernel():
  mesh = plsc.VectorSubcoreMesh(core_axis_name="c", subcore_axis_name="s")

  @functools.partial(
      pl.kernel,
      mesh=mesh,
      out_type=jax.ShapeDtypeStruct((S, B, 2 * D_MODEL), jnp.float32),
      scratch_types=[
          pltpu.VMEM((UPW, IDX_W), jnp.int32),
          pltpu.VMEM((IDX_W, D_MODEL), jnp.float32),
          pltpu.VMEM((IDX_W, D_MODEL), jnp.float32),
          pltpu.VMEM((IDX_W, D_MODEL), jnp.float32),
          pltpu.VMEM((IDX_W, D_MODEL), jnp.float32),
          pltpu.SemaphoreType.DMA,
          pltpu.SemaphoreType.DMA,
          pltpu.SemaphoreType.DMA,
          pltpu.SemaphoreType.DMA,
          pltpu.SemaphoreType.DMA,
          pltpu.SemaphoreType.DMA,
          pltpu.SemaphoreType.DMA,
          pltpu.SemaphoreType.DMA,
      ],
      compiler_params=pltpu.CompilerParams(use_tc_tiling_on_sc=False),
  )
  def emb_kernel(xt_hbm, table_hbm, out_hbm, idx_v, b0, b1, b2, b3,
                 g0, g1, g2, g3, o0, o1, o2, o3):
    bufs = (b0, b1, b2, b3)
    gsems = (g0, g1, g2, g3)
    osems = (o0, o1, o2, o3)
    wid = lax.axis_index("s") * NC + lax.axis_index("c")
    ubase = wid * UPW
    pltpu.sync_copy(xt_hbm.at[pl.ds(ubase, UPW)], idx_v)

    def unit_j_ti(u):
      return (u // (8 * TI)) * 8 + lax.rem(u, 8), lax.rem(u // 8, TI)

    def fire_gather(lu, buf, sem):
      return pltpu.async_copy(table_hbm.at[idx_v.at[lu]], buf, sem)

    def drain_gather(buf, sem):
      pltpu.make_async_copy(table_hbm.at[idx_v.at[0]], buf, sem).wait()

    def scale(buf):
      @plsc.parallel_loop(0, IDX_W, step=1, unroll=8)
      def _(r):
        for cc in range(VPR):
          sl = pl.ds(cc * L, L)
          buf[r, sl] = buf[r, sl] * SCALE

    def fire_out(lu, buf, sem):
      j, ti = unit_j_ti(ubase + lu)
      return pltpu.async_copy(
          buf,
          out_hbm.at[j, pl.ds(ti * IDX_W, IDX_W), pl.ds(0, D_MODEL)],
          sem)

    def drain_out(buf, sem):
      pltpu.make_async_copy(
          buf, out_hbm.at[0, pl.ds(0, IDX_W), pl.ds(0, D_MODEL)], sem).wait()

    for p in range(LOOK):
      fire_gather(p, bufs[p], gsems[p])

    def body(i, carry):
      for p in range(NBUF):
        lu = i * NBUF + p
        pn = (p + LOOK) % NBUF
        if p + LOOK < NBUF:
          @pl.when(i > 0)
          def _():
            drain_out(bufs[pn], osems[pn])
            fire_gather(lu + LOOK, bufs[pn], gsems[pn])

          @pl.when(i == 0)
          def _():
            fire_gather(lu + LOOK, bufs[pn], gsems[pn])
        else:
          @pl.when(i < UPW // NBUF - 1)
          def _():
            drain_out(bufs[pn], osems[pn])
            fire_gather(lu + LOOK, bufs[pn], gsems[pn])
        drain_gather(bufs[p], gsems[p])
        scale(bufs[p])
        fire_out(lu, bufs[p], osems[p])
      return carry

    lax.fori_loop(0, UPW // NBUF, body, 0)
    for p in range(NBUF):
      drain_out(bufs[p], osems[p])

  return emb_kernel


_emb_kernel = _make_kernel()


@jax.jit
def kernel(x, table):
  xt = (
      x.astype(jnp.int32)
      .T.reshape(25, 8, TI, IDX_W)
      .transpose(0, 2, 1, 3)
      .reshape(N_UNITS, IDX_W)
  )
  out = _emb_kernel(xt, table)
  return out[:, :, :D_MODEL].transpose(1, 0, 2)
